# Initial kernel scaffold; baseline (speedup 1.0000x reference)
import jax
import jax.numpy as jnp
from jax.experimental import pallas as pl


def kernel(node_attrs, node_feats, edge_attrs, edge_feats, edge_index, W_up, W1, W2, W3, W4, W_lin, W_skip):
    raise NotImplementedError("write your pallas kernel here")


# speedup vs baseline: 2.1087x; 2.1087x over previous
import functools
import math

import jax
import jax.numpy as jnp
from jax import lax
from jax.experimental import pallas as pl
from jax.experimental.pallas import tpu as pltpu
from jax.experimental.pallas import tpu_sc as plsc

N = 10000
E = 320000
D = 128
A = 10
R = 8
H = 64
AVG_NUM_NEIGHBORS = 32.0

NC = 2
NS = 16
NW = NC * NS
EPW = E // NW
CHUNK = 80
NCHUNK = EPW // CHUNK
NP = 10240
ROWS_PT = NP // NS
ROWBLK = 128

INV_SQRT_R = 1.0 / math.sqrt(R)
INV_SQRT_H = 1.0 / math.sqrt(H)
INV_SQRT_D = 1.0 / math.sqrt(D)
INV_SQRT_DA = 1.0 / math.sqrt(D * A)


def _edge_mlp_body(ef_ref, ea_ref, w1_ref, w2_ref, w3_ref, w4_ref, out_ref):
    h = jax.nn.silu(
        jnp.dot(ef_ref[...], w1_ref[...], preferred_element_type=jnp.float32)
        * INV_SQRT_R)
    h = jax.nn.silu(
        jnp.dot(h, w2_ref[...], preferred_element_type=jnp.float32)
        * INV_SQRT_H)
    h = jax.nn.silu(
        jnp.dot(h, w3_ref[...], preferred_element_type=jnp.float32)
        * INV_SQRT_H)
    tpw = (jnp.dot(h, w4_ref[...], preferred_element_type=jnp.float32)
           * INV_SQRT_H)
    out_ref[...] = tpw * ea_ref[...]


def _edge_mlp(edge_feats, edge_attrs, W1, W2, W3, W4):
    EB = 2560
    grid = (E // EB,)
    return pl.pallas_call(
        _edge_mlp_body,
        grid=grid,
        in_specs=[
            pl.BlockSpec((EB, R), lambda i: (i, 0)),
            pl.BlockSpec((EB, 1), lambda i: (i, 0)),
            pl.BlockSpec((R, H), lambda i: (0, 0)),
            pl.BlockSpec((H, H), lambda i: (0, 0)),
            pl.BlockSpec((H, H), lambda i: (0, 0)),
            pl.BlockSpec((H, D), lambda i: (0, 0)),
        ],
        out_specs=pl.BlockSpec((EB, D), lambda i: (i, 0)),
        out_shape=jax.ShapeDtypeStruct((E, D), jnp.float32),
    )(edge_feats, edge_attrs, W1, W2, W3, W4)


def _up_body(nf_ref, w_ref, out_ref):
    out_ref[...] = (jnp.dot(nf_ref[...], w_ref[...],
                            preferred_element_type=jnp.float32) * INV_SQRT_D)


def _linear_up(node_feats, W_up):
    return pl.pallas_call(
        _up_body,
        out_shape=jax.ShapeDtypeStruct((N, D), jnp.float32),
    )(node_feats, W_up)


def _sc_edge_body(x_hbm, tpw_hbm, snd_hbm, rcv_hbm, out_hbm,
                  idx_v, ridx_v, tpw_v, xrows_v, rowbuf_v, acc_sh, sem):
    c = lax.axis_index("c")
    s = lax.axis_index("s")
    wid = s * NC + c

    zero16 = jnp.zeros((16,), jnp.float32)

    def zero_row(r, _):
        for j in range(D // 16):
            rowbuf_v[r, pl.ds(j * 16, 16)] = zero16
        return 0

    lax.fori_loop(0, ROWBLK, zero_row, 0)

    def zero_acc(k, _):
        r0 = s * ROWS_PT + k * ROWBLK
        pltpu.sync_copy(rowbuf_v, acc_sh.at[pl.ds(r0, ROWBLK)])
        return 0

    lax.fori_loop(0, ROWS_PT // ROWBLK, zero_acc, 0)
    plsc.subcore_barrier()

    def chunk(i, _):
        e0 = wid * EPW + i * CHUNK
        pltpu.sync_copy(snd_hbm.at[pl.ds(e0, CHUNK)], idx_v)
        pltpu.async_copy(x_hbm.at[idx_v], xrows_v, sem).wait()
        pltpu.sync_copy(tpw_hbm.at[pl.ds(e0, CHUNK)], tpw_v)

        def mul_row(r, _):
            for j in range(D // 16):
                sl = pl.ds(j * 16, 16)
                xrows_v[r, sl] = xrows_v[r, sl] * tpw_v[r, sl]
            return 0

        lax.fori_loop(0, CHUNK, mul_row, 0)

        pltpu.sync_copy(rcv_hbm.at[pl.ds(e0, CHUNK)], ridx_v)
        pltpu.sync_copy(xrows_v, acc_sh.at[ridx_v], add=True)
        return 0

    lax.fori_loop(0, NCHUNK, chunk, 0)
    plsc.subcore_barrier()

    def writeout(k, _):
        r0 = s * ROWS_PT + k * ROWBLK
        pltpu.sync_copy(acc_sh.at[pl.ds(r0, ROWBLK)], rowbuf_v)
        pltpu.sync_copy(rowbuf_v, out_hbm.at[pl.ds(c * NP + r0, ROWBLK)])
        return 0

    lax.fori_loop(0, ROWS_PT // ROWBLK, writeout, 0)


def _sc_edge(x, tpw, snd, rcv):
    mesh = plsc.VectorSubcoreMesh(core_axis_name="c", subcore_axis_name="s")
    fn = functools.partial(
        pl.kernel,
        mesh=mesh,
        out_type=jax.ShapeDtypeStruct((NC * NP, D), jnp.float32),
        scratch_types=[
            pltpu.VMEM((CHUNK,), jnp.int32),
            pltpu.VMEM((CHUNK,), jnp.int32),
            pltpu.VMEM((CHUNK, D), jnp.float32),
            pltpu.VMEM((CHUNK, D), jnp.float32),
            pltpu.VMEM((ROWBLK, D), jnp.float32),
            pltpu.VMEM_SHARED((NP, D), jnp.float32),
            pltpu.SemaphoreType.DMA,
        ],
    )(_sc_edge_body)
    return fn(x, tpw, snd, rcv)


def _post_body(p_ref, na_ref, wlin_ref, wsk_ref, out_ref):
    msg = p_ref[0] + p_ref[1]
    msg = (jnp.dot(msg, wlin_ref[...], preferred_element_type=jnp.float32)
           * (INV_SQRT_D / AVG_NUM_NEIGHBORS))
    acc = jnp.zeros_like(out_ref)
    for v in range(A):
        acc += na_ref[:, v:v + 1] * jnp.dot(
            msg, wsk_ref[v], preferred_element_type=jnp.float32)
    out_ref[...] = acc * INV_SQRT_DA


def _post(parts, node_attrs, W_lin, W_skip_t):
    NB = 1000
    grid = (N // NB,)
    return pl.pallas_call(
        _post_body,
        grid=grid,
        in_specs=[
            pl.BlockSpec((NC, NB, D), lambda i: (0, i, 0)),
            pl.BlockSpec((NB, A), lambda i: (i, 0)),
            pl.BlockSpec((D, D), lambda i: (0, 0)),
            pl.BlockSpec((A, D, D), lambda i: (0, 0, 0)),
        ],
        out_specs=pl.BlockSpec((NB, D), lambda i: (i, 0)),
        out_shape=jax.ShapeDtypeStruct((N, D), jnp.float32),
    )(parts, node_attrs, W_lin, W_skip_t)


def kernel(node_attrs, node_feats, edge_attrs, edge_feats, edge_index,
           W_up, W1, W2, W3, W4, W_lin, W_skip):
    snd = edge_index[0].astype(jnp.int32)
    rcv = edge_index[1].astype(jnp.int32)
    tpw = _edge_mlp(edge_feats, edge_attrs, W1, W2, W3, W4)
    x = _linear_up(node_feats, W_up)
    parts = _sc_edge(x, tpw, snd, rcv).reshape(NC, NP, D)[:, :N]
    return _post(parts, node_attrs, W_lin, W_skip.transpose(1, 0, 2))
